# baseline (speedup 1.0000x reference)
import jax
import jax.numpy as jnp
from jax import lax
from jax.experimental import pallas as pl
from jax.experimental.pallas import tpu as pltpu
from jax.experimental.pallas import tpu_sc as plsc

D_MODEL = 2048
B_TOTAL = 4 * 8192
NUM_CORES = 2
NUM_SUBCORES = 16
NW = NUM_CORES * NUM_SUBCORES
B_PER_W = B_TOTAL // NW
CHUNK = 16
NCH = B_PER_W // CHUNK


def _gather_body(idx_hbm, table_hbm, out_hbm, idx_v, buf0, buf1, g0, g1, osem):
    wid = lax.axis_index("s") * NUM_CORES + lax.axis_index("c")
    chunk0 = wid * NCH
    pltpu.sync_copy(idx_hbm.at[pl.ds(chunk0, NCH)], idx_v)

    def gstart(j, buf, sem):
        pltpu.async_copy(table_hbm.at[idx_v.at[j]], buf, sem)

    def gwait(j, buf, sem):
        pltpu.make_async_copy(table_hbm.at[idx_v.at[j]], buf, sem).wait()

    def out(j, buf):
        pltpu.async_copy(buf, out_hbm.at[pl.ds((chunk0 + j) * CHUNK, CHUNK)],
                         osem).wait()

    gstart(0, buf0, g0)
    gstart(1, buf1, g1)

    def body(i, carry):
        j0 = 2 * i
        gwait(j0, buf0, g0)
        out(j0, buf0)
        gstart(j0 + 2, buf0, g0)
        j1 = j0 + 1
        gwait(j1, buf1, g1)
        out(j1, buf1)
        gstart(j1 + 2, buf1, g1)
        return carry

    lax.fori_loop(0, NCH // 2 - 1, body, 0)

    gwait(NCH - 2, buf0, g0)
    out(NCH - 2, buf0)
    gwait(NCH - 1, buf1, g1)
    out(NCH - 1, buf1)


@jax.jit
def _run(idx2d, table):
    return pl.kernel(
        _gather_body,
        out_type=jax.ShapeDtypeStruct((B_TOTAL, D_MODEL), jnp.float32),
        mesh=plsc.VectorSubcoreMesh(core_axis_name="c", subcore_axis_name="s"),
        scratch_types=[
            pltpu.VMEM((NCH, CHUNK), jnp.int32),
            pltpu.VMEM((CHUNK, D_MODEL), jnp.float32),
            pltpu.VMEM((CHUNK, D_MODEL), jnp.float32),
            pltpu.SemaphoreType.DMA,
            pltpu.SemaphoreType.DMA,
            pltpu.SemaphoreType.DMA,
        ],
    )(idx2d, table)


def kernel(thought_ids, thought_embeddings):
    batch_shape = thought_ids.shape
    idx2d = jnp.asarray(thought_ids, jnp.int32).reshape(B_TOTAL // CHUNK, CHUNK)
    out = _run(idx2d, thought_embeddings)
    return out.reshape(*batch_shape, D_MODEL)
